# baseline (speedup 1.0000x reference)
import jax
import jax.numpy as jnp
from jax import lax
from jax.experimental import pallas as pl
from jax.experimental.pallas import tpu as pltpu
from jax.experimental.pallas import tpu_sc as plsc

AUTHOR_VOCAB = 1000000
SUBREDDIT_VOCAB = 100000
EMBED_DIM = 20
EMBED_PAD = 24
BATCH = 16384
NROWS = 2 * BATCH

NC = 2
NS = 16
NW = NC * NS
R_PER_W = NROWS // NW
CHUNK = 128
NCHUNK = R_PER_W // CHUNK

_mesh = plsc.VectorSubcoreMesh(core_axis_name="c", subcore_axis_name="s",
                               num_cores=NC)


def _body(ids_hbm, tab_hbm, out_hbm, idx_v, rows_v, sem):
    wid = lax.axis_index("s") * NC + lax.axis_index("c")

    pltpu.sync_copy(ids_hbm.at[pl.ds(wid * NCHUNK, NCHUNK)], idx_v)

    copies = []
    for j in range(NCHUNK):
        copies.append(pltpu.async_copy(
            tab_hbm.at[idx_v.at[j]],
            rows_v.at[pl.ds(j * CHUNK, CHUNK)], sem))
    for c in copies:
        c.wait()

    pltpu.sync_copy(rows_v, out_hbm.at[pl.ds(wid * R_PER_W, R_PER_W)])


_gather_concat = pl.kernel(
    _body,
    mesh=_mesh,
    out_type=jax.ShapeDtypeStruct((NROWS, EMBED_PAD), jnp.float32),
    scratch_types=[
        pltpu.VMEM((NCHUNK, CHUNK), jnp.int32),
        pltpu.VMEM((R_PER_W, EMBED_PAD), jnp.float32),
        pltpu.SemaphoreType.DMA,
    ],
    compiler_params=pltpu.CompilerParams(use_tc_tiling_on_sc=False),
)


def kernel(author_ids, subreddit_ids, author_table, subreddit_table):
    pad = ((0, 0), (0, EMBED_PAD - EMBED_DIM))
    tab = jnp.concatenate(
        [jnp.pad(author_table, pad), jnp.pad(subreddit_table, pad)], axis=0)
    tab = tab + jnp.float32(0.0)
    ids = jnp.stack(
        [author_ids.astype(jnp.int32),
         subreddit_ids.astype(jnp.int32) + AUTHOR_VOCAB],
        axis=1).reshape(NROWS // CHUNK, CHUNK)
    out2 = _gather_concat(ids, tab)
    return out2[:, :EMBED_DIM].reshape(BATCH, 2 * EMBED_DIM)


# speedup vs baseline: 1.0014x; 1.0014x over previous
import jax
import jax.numpy as jnp
from jax import lax
from jax.experimental import pallas as pl
from jax.experimental.pallas import tpu as pltpu
from jax.experimental.pallas import tpu_sc as plsc

AUTHOR_VOCAB = 1000000
SUBREDDIT_VOCAB = 100000
EMBED_DIM = 20
EMBED_PAD = 24
BATCH = 16384
NROWS = 2 * BATCH

NC = 2
NS = 16
NW = NC * NS
R_PER_W = NROWS // NW
CHUNK = 128
NCHUNK = R_PER_W // CHUNK

_mesh = plsc.VectorSubcoreMesh(core_axis_name="c", subcore_axis_name="s",
                               num_cores=NC)


def _body(ids_hbm, tab_hbm, out_hbm, idx_v, rows_v, sem):
    wid = lax.axis_index("s") * NC + lax.axis_index("c")

    pltpu.sync_copy(ids_hbm.at[pl.ds(wid * NCHUNK, NCHUNK)], idx_v)

    copies = []
    for j in range(NCHUNK):
        copies.append(pltpu.async_copy(
            tab_hbm.at[idx_v.at[j]],
            rows_v.at[pl.ds(j * CHUNK, CHUNK)], sem))
    for c in copies:
        c.wait()

    pltpu.sync_copy(rows_v, out_hbm.at[pl.ds(wid * R_PER_W, R_PER_W)])


_gather_concat = pl.kernel(
    _body,
    mesh=_mesh,
    out_type=jax.ShapeDtypeStruct((NROWS, EMBED_PAD), jnp.float32),
    scratch_types=[
        pltpu.VMEM((NCHUNK, CHUNK), jnp.int32),
        pltpu.VMEM((R_PER_W, EMBED_PAD), jnp.float32),
        pltpu.SemaphoreType.DMA,
    ],
    compiler_params=pltpu.CompilerParams(use_tc_tiling_on_sc=False),
)


def kernel(author_ids, subreddit_ids, author_table, subreddit_table):
    pad = ((0, 0), (0, EMBED_PAD - EMBED_DIM))
    tab = jnp.concatenate(
        [jnp.pad(author_table, pad), jnp.pad(subreddit_table, pad)], axis=0)
    ids = jnp.stack(
        [author_ids.astype(jnp.int32),
         subreddit_ids.astype(jnp.int32) + AUTHOR_VOCAB],
        axis=1).reshape(NROWS // CHUNK, CHUNK)
    out2 = _gather_concat(ids, tab)
    return out2[:, :EMBED_DIM].reshape(BATCH, 2 * EMBED_DIM)
